# baseline (speedup 1.0000x reference)
import functools

import jax
import jax.numpy as jnp
from jax import lax
from jax.experimental import pallas as pl
from jax.experimental.pallas import tpu as pltpu
from jax.experimental.pallas import tpu_sc as plsc

N_NODES = 50000
NUM_GRAPHS = 256
D_FEAT = 256

NS = 16
LANES = 16

N_PER_W = 3136
W_LAST = NS - 1
TAIL = N_NODES - W_LAST * N_PER_W

INTERLEAVE = 4
NB = NUM_GRAPHS + 2 * LANES


def _sc_counts_body(gid_hbm, out_hbm, ids_v, lb_v, hist_v, cnt_v):
    w = lax.axis_index("s")
    base = w * N_PER_W
    n_here = jnp.where(w < W_LAST, N_PER_W, TAIL)

    pltpu.sync_copy(gid_hbm, ids_v)

    g_lo = jnp.min(ids_v[pl.ds(base, LANES)])
    g_hi = jnp.max(ids_v[pl.ds(base + n_here - LANES, LANES)])
    k_lo = lax.shift_right_logical(g_lo, 4)
    k_hi = lax.shift_right_logical(g_hi + 1, 4)

    def chunk_body(k, carry):
        g = k * LANES + lax.iota(jnp.int32, LANES)
        lo0 = jnp.full((LANES,), -1, jnp.int32)
        hi0 = jnp.full((LANES,), N_NODES, jnp.int32)

        def step(_, lohi):
            lo, hi = lohi
            mid = jnp.maximum(lax.shift_right_arithmetic(lo + hi, 1), 0)
            v = plsc.load_gather(ids_v, [mid])
            take_hi = v >= g
            return (jnp.where(take_hi, lo, mid),
                    jnp.where(take_hi, mid, hi))

        _, hi = lax.fori_loop(0, 16, step, (lo0, hi0))
        lb_v[pl.ds(k * LANES, LANES)] = hi
        return carry
    lax.fori_loop(k_lo, k_hi + 2, chunk_body, 0)

    def hist_body(k, carry):
        a = lb_v[pl.ds(k * LANES, LANES)]
        b = lb_v[pl.ds(k * LANES + 1, LANES)]
        hist_v[pl.ds(k * LANES, LANES)] = (b - a).astype(jnp.float32)
        return carry
    lax.fori_loop(k_lo, k_hi + 1, hist_body, 0)

    def gath_body(k, carry):
        for u in range(INTERLEAVE):
            off = (k * INTERLEAVE + u) * LANES
            iv = ids_v[pl.ds(base + off, LANES)]
            cnt_v[pl.ds(off, LANES)] = plsc.load_gather(hist_v, [iv])
        return carry

    @pl.when(w < W_LAST)
    def _():
        lax.fori_loop(0, N_PER_W // (LANES * INTERLEAVE), gath_body, 0)
        pltpu.sync_copy(cnt_v, out_hbm.at[pl.ds(w * N_PER_W, N_PER_W)])

    @pl.when(w == W_LAST)
    def _():
        lax.fori_loop(0, TAIL // (LANES * INTERLEAVE), gath_body, 0)
        iv = ids_v[pl.ds(base + TAIL - LANES, LANES)]
        cnt_v[pl.ds(TAIL - LANES, LANES)] = plsc.load_gather(hist_v, [iv])
        pltpu.sync_copy(cnt_v.at[pl.ds(0, TAIL)],
                        out_hbm.at[pl.ds(w * N_PER_W, TAIL)])


_sc_counts = functools.partial(
    pl.kernel,
    out_type=jax.ShapeDtypeStruct((N_NODES,), jnp.float32),
    mesh=plsc.VectorSubcoreMesh(core_axis_name="c", subcore_axis_name="s", num_cores=1),
    compiler_params=pltpu.CompilerParams(needs_layout_passes=False),
    scratch_types=[
        pltpu.VMEM((N_NODES,), jnp.int32),
        pltpu.VMEM((NB,), jnp.int32),
        pltpu.VMEM((NB,), jnp.float32),
        pltpu.VMEM((N_PER_W,), jnp.float32),
    ],
)(_sc_counts_body)


def _tc_scale_body(feat_ref, cnt_ref, out_ref):
    inv = 1.0 / jnp.sqrt(cnt_ref[...].reshape(ROW_BLOCK, 1))
    out_ref[...] = feat_ref[...] * inv


ROW_BLOCK = 4096


def kernel(feature, graph_node_id):
    gid = graph_node_id.astype(jnp.int32)
    counts = _sc_counts(gid)

    grid = (N_NODES + ROW_BLOCK - 1) // ROW_BLOCK
    return pl.pallas_call(
        _tc_scale_body,
        grid=(grid,),
        in_specs=[
            pl.BlockSpec((ROW_BLOCK, D_FEAT), lambda i: (i, 0)),
            pl.BlockSpec((ROW_BLOCK,), lambda i: (i,)),
        ],
        out_specs=pl.BlockSpec((ROW_BLOCK, D_FEAT), lambda i: (i, 0)),
        out_shape=jax.ShapeDtypeStruct((N_NODES, D_FEAT), jnp.float32),
    )(feature, counts)


# speedup vs baseline: 1.0221x; 1.0221x over previous
import functools

import jax
import jax.numpy as jnp
from jax import lax
from jax.experimental import pallas as pl
from jax.experimental.pallas import tpu as pltpu
from jax.experimental.pallas import tpu_sc as plsc

N_NODES = 50000
NUM_GRAPHS = 256
D_FEAT = 256

NS = 16
LANES = 16

N_PER_W = 3136
W_LAST = NS - 1
TAIL = N_NODES - W_LAST * N_PER_W

INTERLEAVE = 4
NB = NUM_GRAPHS + 2 * LANES


def _sc_counts_body(gid_hbm, out_hbm, ids_v, lb_v, hist_v, cnt_v):
    w = lax.axis_index("s")
    base = w * N_PER_W
    n_here = jnp.where(w < W_LAST, N_PER_W, TAIL)

    pltpu.sync_copy(gid_hbm, ids_v)

    g_lo = jnp.min(ids_v[pl.ds(base, LANES)])
    g_hi = jnp.max(ids_v[pl.ds(base + n_here - LANES, LANES)])
    k_lo = lax.shift_right_logical(g_lo, 4)
    k_hi = lax.shift_right_logical(g_hi + 1, 4)

    def chunk_body(k, carry):
        g = k * LANES + lax.iota(jnp.int32, LANES)
        lo0 = jnp.full((LANES,), -1, jnp.int32)
        hi0 = jnp.full((LANES,), N_NODES, jnp.int32)

        def step(_, lohi):
            lo, hi = lohi
            mid = jnp.maximum(lax.shift_right_arithmetic(lo + hi, 1), 0)
            v = plsc.load_gather(ids_v, [mid])
            take_hi = v >= g
            return (jnp.where(take_hi, lo, mid),
                    jnp.where(take_hi, mid, hi))

        _, hi = lax.fori_loop(0, 16, step, (lo0, hi0))
        lb_v[pl.ds(k * LANES, LANES)] = hi
        return carry
    lax.fori_loop(k_lo, k_hi + 2, chunk_body, 0)

    def hist_body(k, carry):
        a = lb_v[pl.ds(k * LANES, LANES)]
        b = lb_v[pl.ds(k * LANES + 1, LANES)]
        hist_v[pl.ds(k * LANES, LANES)] = (b - a).astype(jnp.float32)
        return carry
    lax.fori_loop(k_lo, k_hi + 1, hist_body, 0)

    def gath_body(k, carry):
        for u in range(INTERLEAVE):
            off = (k * INTERLEAVE + u) * LANES
            iv = ids_v[pl.ds(base + off, LANES)]
            cnt_v[pl.ds(off, LANES)] = plsc.load_gather(hist_v, [iv])
        return carry

    @pl.when(w < W_LAST)
    def _():
        lax.fori_loop(0, N_PER_W // (LANES * INTERLEAVE), gath_body, 0)
        pltpu.sync_copy(cnt_v, out_hbm.at[pl.ds(w * N_PER_W, N_PER_W)])

    @pl.when(w == W_LAST)
    def _():
        lax.fori_loop(0, TAIL // (LANES * INTERLEAVE), gath_body, 0)
        iv = ids_v[pl.ds(base + TAIL - LANES, LANES)]
        cnt_v[pl.ds(TAIL - LANES, LANES)] = plsc.load_gather(hist_v, [iv])
        pltpu.sync_copy(cnt_v.at[pl.ds(0, TAIL)],
                        out_hbm.at[pl.ds(w * N_PER_W, TAIL)])


_sc_counts = functools.partial(
    pl.kernel,
    out_type=jax.ShapeDtypeStruct((N_NODES,), jnp.float32),
    mesh=plsc.VectorSubcoreMesh(core_axis_name="c", subcore_axis_name="s", num_cores=1),
    compiler_params=pltpu.CompilerParams(needs_layout_passes=False),
    scratch_types=[
        pltpu.VMEM((N_NODES,), jnp.int32),
        pltpu.VMEM((NB,), jnp.int32),
        pltpu.VMEM((NB,), jnp.float32),
        pltpu.VMEM((N_PER_W,), jnp.float32),
    ],
)(_sc_counts_body)


def _tc_scale_body(feat_ref, cnt_ref, out_ref):
    inv = 1.0 / jnp.sqrt(cnt_ref[...].reshape(ROW_BLOCK, 1))
    out_ref[...] = feat_ref[...] * inv


ROW_BLOCK = 8192


def kernel(feature, graph_node_id):
    gid = graph_node_id.astype(jnp.int32)
    counts = _sc_counts(gid)

    grid = (N_NODES + ROW_BLOCK - 1) // ROW_BLOCK
    return pl.pallas_call(
        _tc_scale_body,
        grid=(grid,),
        in_specs=[
            pl.BlockSpec((ROW_BLOCK, D_FEAT), lambda i: (i, 0)),
            pl.BlockSpec((ROW_BLOCK,), lambda i: (i,)),
        ],
        out_specs=pl.BlockSpec((ROW_BLOCK, D_FEAT), lambda i: (i, 0)),
        out_shape=jax.ShapeDtypeStruct((N_NODES, D_FEAT), jnp.float32),
    )(feature, counts)
